# Initial kernel scaffold; baseline (speedup 1.0000x reference)
import jax
import jax.numpy as jnp
from jax.experimental import pallas as pl


def kernel(response, emb_table, W, b):
    raise NotImplementedError("write your pallas kernel here")


# speedup vs baseline: 51.5069x; 51.5069x over previous
import functools

import jax
import jax.numpy as jnp
from jax import lax
from jax.experimental import pallas as pl
from jax.experimental.pallas import tpu as pltpu
from jax.experimental.pallas import tpu_sc as plsc

VOCAB = 10000
EMB = 256
B = 4096
L = 200

_INFO = plsc.get_sparse_core_info()
NC = _INFO.num_cores
NS = _INFO.num_subcores
LANES = _INFO.num_lanes
NW = NC * NS
G_TOTAL = B // LANES
G_PER_W = G_TOTAL // NW
IDX_PER_W = L * G_PER_W * LANES

_PROJ_BLOCK = 2000


def _proj_body(emb_ref, w_ref, out_ref):
    out_ref[:] = jnp.sum(emb_ref[:] * w_ref[:], axis=1, keepdims=True) * (1.0 / L)


_proj_call = pl.pallas_call(
    _proj_body,
    grid=(VOCAB // _PROJ_BLOCK,),
    in_specs=[
        pl.BlockSpec((_PROJ_BLOCK, EMB), lambda i: (i, 0)),
        pl.BlockSpec((1, EMB), lambda i: (0, 0)),
    ],
    out_specs=pl.BlockSpec((_PROJ_BLOCK, 1), lambda i: (i, 0)),
    out_shape=jax.ShapeDtypeStruct((VOCAB, 1), jnp.float32),
)


def _sc_body(proj_hbm, idx_hbm, b_hbm, out_hbm, proj_v, idx_v, b_v, out_v):
    wid = lax.axis_index("s") * NC + lax.axis_index("c")
    pltpu.sync_copy(proj_hbm, proj_v)
    pltpu.sync_copy(idx_hbm.at[wid], idx_v)
    pltpu.sync_copy(b_hbm, b_v)
    bvec = b_v[...]

    def body(l, accs):
        base = l * (G_PER_W * LANES)
        return tuple(
            acc + plsc.load_gather(proj_v, [idx_v[pl.ds(base + g * LANES, LANES)]])
            for g, acc in enumerate(accs)
        )

    accs = lax.fori_loop(
        0, L, body, tuple(jnp.zeros((LANES,), jnp.float32) for _ in range(G_PER_W))
    )
    for g in range(G_PER_W):
        out_v[g, :] = accs[g] + bvec
    pltpu.sync_copy(out_v, out_hbm.at[pl.ds(wid * G_PER_W, G_PER_W)])


_sc_call = pl.kernel(
    _sc_body,
    out_type=jax.ShapeDtypeStruct((G_TOTAL, LANES), jnp.float32),
    mesh=plsc.VectorSubcoreMesh(core_axis_name="c", subcore_axis_name="s"),
    compiler_params=pltpu.CompilerParams(needs_layout_passes=False),
    scratch_types=[
        pltpu.VMEM((VOCAB,), jnp.float32),
        pltpu.VMEM((IDX_PER_W,), jnp.int32),
        pltpu.VMEM((LANES,), jnp.float32),
        pltpu.VMEM((G_PER_W, LANES), jnp.float32),
    ],
)


@jax.jit
def kernel(response, emb_table, W, b):
    proj = _proj_call(emb_table, W).reshape(VOCAB)
    idx = (
        response.reshape(NW, G_PER_W, LANES, L)
        .transpose(0, 3, 1, 2)
        .reshape(NW, IDX_PER_W)
    )
    b16 = jnp.broadcast_to(b, (LANES,)).astype(jnp.float32)
    out = _sc_call(proj, idx, b16)
    return out.reshape(B, 1)


# speedup vs baseline: 64.4791x; 1.2519x over previous
import jax
import jax.numpy as jnp
from jax import lax
from jax.experimental import pallas as pl
from jax.experimental.pallas import tpu as pltpu
from jax.experimental.pallas import tpu_sc as plsc

VOCAB = 10000
EMB = 256
B = 4096
L = 200

_INFO = plsc.get_sparse_core_info()
NC = _INFO.num_cores
NS = _INFO.num_subcores
LANES = _INFO.num_lanes
NW = NC * NS
G_TOTAL = B // LANES
G_PER_W = G_TOTAL // NW
IDX_PER_W = L * G_PER_W * LANES

_PROJ_BLOCK = 2000


def _proj_body(emb_ref, w_ref, out_ref):
    out_ref[:] = jnp.sum(emb_ref[:] * w_ref[:], axis=1, keepdims=True) * (1.0 / L)


_proj_call = pl.pallas_call(
    _proj_body,
    grid=(VOCAB // _PROJ_BLOCK,),
    in_specs=[
        pl.BlockSpec((_PROJ_BLOCK, EMB), lambda i: (i, 0)),
        pl.BlockSpec((1, EMB), lambda i: (0, 0)),
    ],
    out_specs=pl.BlockSpec((_PROJ_BLOCK, 1), lambda i: (i, 0)),
    out_shape=jax.ShapeDtypeStruct((VOCAB, 1), jnp.float32),
)


def _sc_body(proj_hbm, resp_hbm, b_hbm, out_hbm, proj_v, resp_v, b_v, out_v):
    wid = lax.axis_index("s") * NC + lax.axis_index("c")
    pltpu.sync_copy(proj_hbm, proj_v)
    pltpu.sync_copy(resp_hbm.at[wid], resp_v)
    pltpu.sync_copy(b_hbm, b_v)
    bvec = b_v[...]
    lane = lax.iota(jnp.int32, 16)
    svecs = [(g * LANES + lane) * L for g in range(G_PER_W)]

    def body(l, accs):
        new = []
        for g in range(G_PER_W):
            tok = plsc.load_gather(resp_v, [svecs[g] + l])
            new.append(accs[g] + plsc.load_gather(proj_v, [tok]))
        return tuple(new)

    accs = lax.fori_loop(
        0, L, body, tuple(jnp.zeros((LANES,), jnp.float32) for _ in range(G_PER_W))
    )
    for g in range(G_PER_W):
        out_v[g, :] = accs[g] + bvec
    pltpu.sync_copy(out_v, out_hbm.at[pl.ds(wid * G_PER_W, G_PER_W)])


_sc_call = pl.kernel(
    _sc_body,
    out_type=jax.ShapeDtypeStruct((G_TOTAL, LANES), jnp.float32),
    mesh=plsc.VectorSubcoreMesh(core_axis_name="c", subcore_axis_name="s"),
    compiler_params=pltpu.CompilerParams(needs_layout_passes=False),
    scratch_types=[
        pltpu.VMEM((VOCAB,), jnp.float32),
        pltpu.VMEM((IDX_PER_W,), jnp.int32),
        pltpu.VMEM((LANES,), jnp.float32),
        pltpu.VMEM((G_PER_W, LANES), jnp.float32),
    ],
)


@jax.jit
def kernel(response, emb_table, W, b):
    proj = _proj_call(emb_table, W).reshape(VOCAB)
    resp = response.reshape(NW, IDX_PER_W)
    b16 = jnp.broadcast_to(b, (LANES,)).astype(jnp.float32)
    out = _sc_call(proj, resp, b16)
    return out.reshape(B, 1)
